# Initial kernel scaffold; baseline (speedup 1.0000x reference)
import jax
import jax.numpy as jnp
from jax.experimental import pallas as pl


def kernel(x, embedding_table):
    raise NotImplementedError("write your pallas kernel here")


# speedup vs baseline: 1.4787x; 1.4787x over previous
import functools

import jax
import jax.numpy as jnp
from jax import lax
from jax.experimental import pallas as pl
from jax.experimental.pallas import tpu as pltpu
from jax.experimental.pallas import tpu_sc as plsc

B = 4096
L = 200
D = 32
NW = 32
B_PER_W = B // NW

IDX_CHUNKS = ((0, 104), (104, 96))

_mesh = plsc.VectorSubcoreMesh(core_axis_name="c", subcore_axis_name="s")


@functools.partial(
    pl.kernel,
    out_type=jax.ShapeDtypeStruct((B, D * L), jnp.float32),
    mesh=_mesh,
    compiler_params=pltpu.CompilerParams(
        needs_layout_passes=False, use_tc_tiling_on_sc=False
    ),
    scratch_types=[
        pltpu.VMEM((L,), jnp.int32),
        pltpu.VMEM((L, D), jnp.float32),
        pltpu.VMEM((D * L,), jnp.float32),
        pltpu.SemaphoreType.DMA,
    ],
)
def _gather_transpose(x_hbm, tab_hbm, out_hbm, idx_v, rows_v, trans_v, sem):
    wid = lax.axis_index("s") * 2 + lax.axis_index("c")
    scat_base = lax.iota(jnp.int32, 16) * L

    def row_body(bl, carry):
        b = wid * B_PER_W + bl
        pltpu.sync_copy(x_hbm.at[b], idx_v)
        copies = [
            pltpu.async_copy(
                tab_hbm.at[idx_v.at[pl.ds(off, n)]],
                rows_v.at[pl.ds(off, n)],
                sem,
            )
            for off, n in IDX_CHUNKS
        ]
        for cp in copies:
            cp.wait()

        def l_body(l, c):
            scat = scat_base + l
            v0 = rows_v[l, pl.ds(0, 16)]
            v1 = rows_v[l, pl.ds(16, 16)]
            plsc.store_scatter(trans_v, [scat], v0)
            plsc.store_scatter(trans_v, [scat + 16 * L], v1)
            return c

        lax.fori_loop(0, L, l_body, 0, unroll=False)
        pltpu.sync_copy(trans_v, out_hbm.at[b])
        return carry

    lax.fori_loop(0, B_PER_W, row_body, 0, unroll=False)


def kernel(x, embedding_table):
    out = _gather_transpose(x.astype(jnp.int32), embedding_table)
    return out.reshape(B, D, L)


# speedup vs baseline: 1.6970x; 1.1476x over previous
import functools

import jax
import jax.numpy as jnp
from jax import lax
from jax.experimental import pallas as pl
from jax.experimental.pallas import tpu as pltpu
from jax.experimental.pallas import tpu_sc as plsc

B = 4096
L = 200
D = 32
NW = 32
B_PER_W = B // NW
G = 4
NG = B_PER_W // G
GL = G * L
GOUT = G * D * L

STREAM_CHUNKS = tuple((o, min(128, GL - o)) for o in range(0, GL, 128))

_mesh = plsc.VectorSubcoreMesh(core_axis_name="c", subcore_axis_name="s")


@functools.partial(
    pl.kernel,
    out_type=jax.ShapeDtypeStruct((B * D * L,), jnp.float32),
    mesh=_mesh,
    compiler_params=pltpu.CompilerParams(
        needs_layout_passes=False, use_tc_tiling_on_sc=False
    ),
    scratch_types=[
        pltpu.VMEM((2, GL), jnp.int32),
        pltpu.VMEM((2, GL, D), jnp.float32),
        pltpu.VMEM((2, GOUT), jnp.float32),
        pltpu.SemaphoreType.DMA,
        pltpu.SemaphoreType.DMA,
        pltpu.SemaphoreType.DMA,
        pltpu.SemaphoreType.DMA,
    ],
)
def _gather_transpose(
    x_hbm, tab_hbm, out_hbm, idx_v, rows_v, trans_v, gsem0, gsem1, osem0, osem1
):
    gsems = (gsem0, gsem1)
    osems = (osem0, osem1)
    wid = lax.axis_index("s") * 2 + lax.axis_index("c")
    w_idx0 = wid * (B_PER_W * L)
    w_out0 = wid * (B_PER_W * D * L)
    scat_base = lax.iota(jnp.int32, 16) * L

    def issue_gather(g, buf):
        pltpu.sync_copy(x_hbm.at[pl.ds(w_idx0 + g * GL, GL)], idx_v.at[buf])
        for off, n in STREAM_CHUNKS:
            pltpu.async_copy(
                tab_hbm.at[idx_v.at[buf, pl.ds(off, n)]],
                rows_v.at[buf, pl.ds(off, n)],
                gsems[buf],
            )

    def drain_gather(buf):
        for off, n in STREAM_CHUNKS:
            pltpu.make_async_copy(
                tab_hbm.at[idx_v.at[buf, pl.ds(off, n)]],
                rows_v.at[buf, pl.ds(off, n)],
                gsems[buf],
            ).wait()

    def transpose(buf):
        @plsc.parallel_loop(0, L, unroll=2)
        def _(l):
            sv = scat_base + l
            for bl in range(G):
                v0 = rows_v[buf, bl * L + l, pl.ds(0, 16)]
                v1 = rows_v[buf, bl * L + l, pl.ds(16, 16)]
                plsc.store_scatter(trans_v.at[buf], [sv + bl * (D * L)], v0)
                plsc.store_scatter(
                    trans_v.at[buf], [sv + bl * (D * L) + 16 * L], v1
                )

    def issue_out(g, buf):
        pltpu.async_copy(
            trans_v.at[buf],
            out_hbm.at[pl.ds(w_out0 + g * GOUT, GOUT)],
            osems[buf],
        )

    def drain_out(g, buf):
        pltpu.make_async_copy(
            trans_v.at[buf],
            out_hbm.at[pl.ds(w_out0 + g * GOUT, GOUT)],
            osems[buf],
        ).wait()

    issue_gather(0, 0)
    issue_gather(1, 1)

    @pl.loop(0, NG, step=2)
    def _(go):
        for sub in (0, 1):
            g = go + sub
            buf = sub
            drain_gather(buf)

            @pl.when(go >= 2)
            def _():
                drain_out(g, buf)

            transpose(buf)
            issue_out(g, buf)

            @pl.when(g + 2 < NG)
            def _():
                issue_gather(g + 2, buf)

    drain_out(NG - 2, 0)
    drain_out(NG - 1, 1)


def kernel(x, embedding_table):
    out = _gather_transpose(x.reshape(-1).astype(jnp.int32), embedding_table)
    return out.reshape(B, D, L)
